# baseline (speedup 1.0000x reference)
import functools

import numpy as np
import jax
import jax.numpy as jnp
from jax import lax
from jax.experimental import pallas as pl
from jax.experimental.pallas import tpu as pltpu
from jax.experimental.pallas import tpu_sc as plsc

_VOCAB = 1000000
_EMBED = 64
_BATCH = 16384
_SLEN = 50

_NC = 2
_NS = 16
_NW = _NC * _NS

_CB = 16
_IDX_MINOR = 100
_ROWS_PER_STEP = _CB * _SLEN
_IDX_ROWS = _ROWS_PER_STEP // _IDX_MINOR
_BPW = _BATCH // _NW
_STEPS = _BPW // _CB


def _pe_weights():
    l = np.array([[1.0 - s / _SLEN - e / _EMBED * (1.0 - 2.0 * s / _SLEN)
                   for s in range(_SLEN)] for e in range(_EMBED)],
                 dtype=np.float32)
    return jnp.asarray(l.reshape(_SLEN, _EMBED))


_TCH = 384
_TCOLS_FULL = (_VOCAB // 128) * 128
_TCHUNKS = _TCOLS_FULL // _TCH
_TPW = _TCHUNKS // _NW
_TEXTRA = _TCHUNKS - _TPW * _NW
_TTAIL = _VOCAB - _TCOLS_FULL


def _transpose(word_embedding):
    t_fm = word_embedding.T
    tail_fm = word_embedding[_TCOLS_FULL:].T
    mesh = plsc.VectorSubcoreMesh(core_axis_name="c", subcore_axis_name="s")

    @functools.partial(
        pl.kernel,
        out_type=jax.ShapeDtypeStruct((_VOCAB // 2, 128), jnp.float32),
        mesh=mesh,
        scratch_types=[
            pltpu.VMEM((_EMBED, _TCH), jnp.float32),
            pltpu.VMEM((_EMBED, _TCH), jnp.float32),
            pltpu.VMEM((_TCH // 2, 128), jnp.float32),
            pltpu.VMEM((_TCH // 2, 128), jnp.float32),
            pltpu.VMEM((_EMBED, _TTAIL), jnp.float32),
            pltpu.SemaphoreType.DMA,
            pltpu.SemaphoreType.DMA,
            pltpu.SemaphoreType.DMA,
            pltpu.SemaphoreType.DMA,
        ],
        compiler_params=pltpu.CompilerParams(use_tc_tiling_on_sc=True,
                                             needs_layout_passes=False),
    )
    def trun(t_hbm, tail_hbm, o_hbm,
             src0, src1, dst0, dst1, tail_v, is0, is1, os0, os1):
        wid = lax.axis_index("s") * _NC + lax.axis_index("c")
        nch = _TPW + jnp.where(wid < _TEXTRA, 1, 0)
        rows = [lax.iota(jnp.int32, 16) + 16 * k for k in range(4)]

        def fire_in(j, src, sem):
            c = wid + j * _NW
            pltpu.async_copy(t_hbm.at[:, pl.ds(c * _TCH, _TCH)], src, sem)

        def wait_in(j, src, sem):
            c = wid + j * _NW
            pltpu.make_async_copy(t_hbm.at[:, pl.ds(c * _TCH, _TCH)],
                                  src, sem).wait()

        def shuffle(src, dst, ncols):
            @pl.loop(0, ncols // 2)
            def _(m):
                for half in range(2):
                    cvec = jnp.full((16,), 2 * m + half, jnp.int32)
                    for k in range(4):
                        v = plsc.load_gather(src, [rows[k], cvec])
                        dst[m, pl.ds(half * 64 + 16 * k, 16)] = v

        def fire_out(j, dst, sem):
            c = wid + j * _NW
            pltpu.async_copy(dst,
                             o_hbm.at[pl.ds(c * (_TCH // 2), _TCH // 2)], sem)

        def wait_out(j, dst, sem):
            c = wid + j * _NW
            pltpu.make_async_copy(
                dst, o_hbm.at[pl.ds(c * (_TCH // 2), _TCH // 2)], sem).wait()

        fire_in(jnp.int32(0), src0, is0)

        @pl.when(nch >= 2)
        def _():
            fire_in(jnp.int32(1), src1, is1)

        @pl.loop(0, _TPW + 1, step=2)
        def _(j0):
            def do(j, src, isem, dst, osem):
                @pl.when(j < nch)
                def _():
                    wait_in(j, src, isem)

                    @pl.when(j >= 2)
                    def _():
                        wait_out(j - 2, dst, osem)

                    shuffle(src, dst, _TCH)
                    fire_out(j, dst, osem)

                    @pl.when(j + 2 < nch)
                    def _():
                        fire_in(j + 2, src, isem)

            do(j0, src0, is0, dst0, os0)
            do(j0 + 1, src1, is1, dst1, os1)

        wait_out(2 * ((nch - 1) // 2), dst0, os0)
        wait_out(2 * ((nch - 2) // 2) + 1, dst1, os1)

        @pl.when(wid == 0)
        def _():
            pltpu.sync_copy(tail_hbm, tail_v)
            @pl.loop(0, _TTAIL // 2)
            def _(m):
                for half in range(2):
                    cvec = jnp.full((16,), 2 * m + half, jnp.int32)
                    for k in range(4):
                        v = plsc.load_gather(tail_v, [rows[k], cvec])
                        dst0[m, pl.ds(half * 64 + 16 * k, 16)] = v
            pltpu.sync_copy(dst0.at[pl.ds(0, _TTAIL // 2)],
                            o_hbm.at[pl.ds(_TCOLS_FULL // 2, _TTAIL // 2)])

    return trun(t_fm, tail_fm)


def kernel(questions, word_embedding):
    q2 = questions.reshape(_BATCH * _SLEN // _IDX_MINOR, _IDX_MINOR)
    t128 = _transpose(word_embedding)
    table_rm = t128.reshape(_VOCAB, _EMBED)
    w = _pe_weights()
    mesh = plsc.VectorSubcoreMesh(core_axis_name="c", subcore_axis_name="s")

    @functools.partial(
        pl.kernel,
        out_type=jax.ShapeDtypeStruct((_BATCH, _EMBED), jnp.float32),
        mesh=mesh,
        scratch_types=[
            pltpu.VMEM((_IDX_ROWS, _IDX_MINOR), jnp.int32),
            pltpu.VMEM((_IDX_ROWS, _IDX_MINOR), jnp.int32),
            pltpu.VMEM((_ROWS_PER_STEP, _EMBED), jnp.float32),
            pltpu.VMEM((_ROWS_PER_STEP, _EMBED), jnp.float32),
            pltpu.VMEM((_SLEN, _EMBED), jnp.float32),
            pltpu.VMEM((_CB, _EMBED), jnp.float32),
            pltpu.SemaphoreType.DMA,
            pltpu.SemaphoreType.DMA,
        ],
        compiler_params=pltpu.CompilerParams(use_tc_tiling_on_sc=False),
    )
    def run(q_hbm, t_hbm, w_hbm, o_hbm,
            idx0, idx1, rows0, rows1, w_v, out_v, sem0, sem1):
        wid = lax.axis_index("s") * _NC + lax.axis_index("c")
        qbase = wid * (_BPW * _SLEN // _IDX_MINOR)
        pltpu.sync_copy(w_hbm, w_v)

        def fire(step, idx_v, rows_v, sem):
            pltpu.sync_copy(q_hbm.at[pl.ds(qbase + step * _IDX_ROWS, _IDX_ROWS)],
                            idx_v)
            for j in range(_IDX_ROWS):
                pltpu.async_copy(
                    t_hbm.at[idx_v.at[j]],
                    rows_v.at[pl.ds(j * _IDX_MINOR, _IDX_MINOR)],
                    sem,
                )

        def drain(idx_v, rows_v, sem):
            for j in range(_IDX_ROWS):
                pltpu.make_async_copy(
                    t_hbm.at[idx_v.at[j]],
                    rows_v.at[pl.ds(j * _IDX_MINOR, _IDX_MINOR)],
                    sem,
                ).wait()

        def compute(step, rows_v):
            for b0 in range(0, _CB, 4):
                def body(l, accs):
                    ws = [w_v[l, pl.ds(16 * k, 16)] for k in range(4)]
                    nxt = []
                    for g in range(4):
                        r = (b0 + g) * _SLEN + l
                        for k in range(4):
                            nxt.append(accs[g * 4 + k]
                                       + rows_v[r, pl.ds(16 * k, 16)] * ws[k])
                    return tuple(nxt)

                zero = jnp.zeros((16,), jnp.float32)
                accs = lax.fori_loop(0, _SLEN, body, (zero,) * 16)
                for g in range(4):
                    for k in range(4):
                        out_v[b0 + g, pl.ds(16 * k, 16)] = accs[g * 4 + k]
            pltpu.sync_copy(out_v,
                            o_hbm.at[pl.ds(wid * _BPW + step * _CB, _CB)])

        fire(jnp.int32(0), idx0, rows0, sem0)

        @pl.loop(0, _STEPS, step=2)
        def _(s0):
            fire(s0 + 1, idx1, rows1, sem1)
            drain(idx0, rows0, sem0)
            compute(s0, rows0)

            @pl.when(s0 + 2 < _STEPS)
            def _():
                fire(s0 + 2, idx0, rows0, sem0)

            drain(idx1, rows1, sem1)
            compute(s0 + 1, rows1)

    return run(q2, table_rm, w)


# speedup vs baseline: 5.0494x; 5.0494x over previous
import functools

import numpy as np
import jax
import jax.numpy as jnp
from jax import lax
from jax.experimental import pallas as pl
from jax.experimental.pallas import tpu as pltpu
from jax.experimental.pallas import tpu_sc as plsc

_VOCAB = 1000000
_EMBED = 64
_BATCH = 16384
_SLEN = 50

_NC = 2
_NS = 16
_NW = _NC * _NS

_CB = 16
_IDX_MINOR = 100
_ROWS_PER_STEP = _CB * _SLEN
_IDX_ROWS = _ROWS_PER_STEP // _IDX_MINOR
_BPW = _BATCH // _NW
_STEPS = _BPW // _CB


def _pe_weights():
    l = np.array([[1.0 - s / _SLEN - e / _EMBED * (1.0 - 2.0 * s / _SLEN)
                   for s in range(_SLEN)] for e in range(_EMBED)],
                 dtype=np.float32)
    return jnp.asarray(l.reshape(_SLEN, _EMBED))


_TCH = 384
_TCOLS_FULL = (_VOCAB // 128) * 128
_TCHUNKS = _TCOLS_FULL // _TCH
_TPW = _TCHUNKS // _NW
_TEXTRA = _TCHUNKS - _TPW * _NW
_TTAIL = _VOCAB - _TCOLS_FULL


def _transpose(word_embedding):
    t_fm = word_embedding.T
    tail_fm = word_embedding[_TCOLS_FULL:].T
    mesh = plsc.VectorSubcoreMesh(core_axis_name="c", subcore_axis_name="s")

    @functools.partial(
        pl.kernel,
        out_type=jax.ShapeDtypeStruct((_VOCAB // 2, 128), jnp.float32),
        mesh=mesh,
        scratch_types=[
            pltpu.VMEM((_EMBED, _TCH), jnp.float32),
            pltpu.VMEM((_EMBED, _TCH), jnp.float32),
            pltpu.VMEM((_TCH // 2, 128), jnp.float32),
            pltpu.VMEM((_TCH // 2, 128), jnp.float32),
            pltpu.VMEM((_EMBED, _TTAIL), jnp.float32),
            pltpu.SemaphoreType.DMA,
            pltpu.SemaphoreType.DMA,
            pltpu.SemaphoreType.DMA,
            pltpu.SemaphoreType.DMA,
        ],
        compiler_params=pltpu.CompilerParams(use_tc_tiling_on_sc=True,
                                             needs_layout_passes=False),
    )
    def trun(t_hbm, tail_hbm, o_hbm,
             src0, src1, dst0, dst1, tail_v, is0, is1, os0, os1):
        wid = lax.axis_index("s") * _NC + lax.axis_index("c")
        nch = _TPW + jnp.where(wid < _TEXTRA, 1, 0)
        rows = [lax.iota(jnp.int32, 16) + 16 * k for k in range(4)]

        def fire_in(j, src, sem):
            c = wid + j * _NW
            pltpu.async_copy(t_hbm.at[:, pl.ds(c * _TCH, _TCH)], src, sem)

        def wait_in(j, src, sem):
            c = wid + j * _NW
            pltpu.make_async_copy(t_hbm.at[:, pl.ds(c * _TCH, _TCH)],
                                  src, sem).wait()

        def shuffle(src, dst, ncols):
            @functools.partial(plsc.parallel_loop, 0, ncols // 2, unroll=1)
            def _(m):
                for half in range(2):
                    cvec = jnp.full((16,), 2 * m + half, jnp.int32)
                    for k in range(4):
                        v = plsc.load_gather(src, [rows[k], cvec])
                        dst[m, pl.ds(half * 64 + 16 * k, 16)] = v

        def fire_out(j, dst, sem):
            c = wid + j * _NW
            pltpu.async_copy(dst,
                             o_hbm.at[pl.ds(c * (_TCH // 2), _TCH // 2)], sem)

        def wait_out(j, dst, sem):
            c = wid + j * _NW
            pltpu.make_async_copy(
                dst, o_hbm.at[pl.ds(c * (_TCH // 2), _TCH // 2)], sem).wait()

        fire_in(jnp.int32(0), src0, is0)

        @pl.when(nch >= 2)
        def _():
            fire_in(jnp.int32(1), src1, is1)

        @pl.loop(0, _TPW + 1, step=2)
        def _(j0):
            def do(j, src, isem, dst, osem):
                @pl.when(j < nch)
                def _():
                    wait_in(j, src, isem)

                    @pl.when(j >= 2)
                    def _():
                        wait_out(j - 2, dst, osem)

                    shuffle(src, dst, _TCH)
                    fire_out(j, dst, osem)

                    @pl.when(j + 2 < nch)
                    def _():
                        fire_in(j + 2, src, isem)

            do(j0, src0, is0, dst0, os0)
            do(j0 + 1, src1, is1, dst1, os1)

        wait_out(2 * ((nch - 1) // 2), dst0, os0)
        wait_out(2 * ((nch - 2) // 2) + 1, dst1, os1)

        @pl.when(wid == 0)
        def _():
            pltpu.sync_copy(tail_hbm, tail_v)
            @functools.partial(plsc.parallel_loop, 0, _TTAIL // 2, unroll=1)
            def _(m):
                for half in range(2):
                    cvec = jnp.full((16,), 2 * m + half, jnp.int32)
                    for k in range(4):
                        v = plsc.load_gather(tail_v, [rows[k], cvec])
                        dst0[m, pl.ds(half * 64 + 16 * k, 16)] = v
            pltpu.sync_copy(dst0.at[pl.ds(0, _TTAIL // 2)],
                            o_hbm.at[pl.ds(_TCOLS_FULL // 2, _TTAIL // 2)])

    return trun(t_fm, tail_fm)


def kernel(questions, word_embedding):
    q2 = questions.reshape(_BATCH * _SLEN // _IDX_MINOR, _IDX_MINOR)
    t128 = _transpose(word_embedding)
    table_rm = t128.reshape(_VOCAB, _EMBED)
    w = _pe_weights()
    mesh = plsc.VectorSubcoreMesh(core_axis_name="c", subcore_axis_name="s")

    @functools.partial(
        pl.kernel,
        out_type=jax.ShapeDtypeStruct((_BATCH, _EMBED), jnp.float32),
        mesh=mesh,
        scratch_types=[
            pltpu.VMEM((_IDX_ROWS, _IDX_MINOR), jnp.int32),
            pltpu.VMEM((_IDX_ROWS, _IDX_MINOR), jnp.int32),
            pltpu.VMEM((_ROWS_PER_STEP, _EMBED), jnp.float32),
            pltpu.VMEM((_ROWS_PER_STEP, _EMBED), jnp.float32),
            pltpu.VMEM((_SLEN, _EMBED), jnp.float32),
            pltpu.VMEM((_CB, _EMBED), jnp.float32),
            pltpu.SemaphoreType.DMA,
            pltpu.SemaphoreType.DMA,
        ],
        compiler_params=pltpu.CompilerParams(use_tc_tiling_on_sc=False),
    )
    def run(q_hbm, t_hbm, w_hbm, o_hbm,
            idx0, idx1, rows0, rows1, w_v, out_v, sem0, sem1):
        wid = lax.axis_index("s") * _NC + lax.axis_index("c")
        qbase = wid * (_BPW * _SLEN // _IDX_MINOR)
        pltpu.sync_copy(w_hbm, w_v)

        def fire(step, idx_v, rows_v, sem):
            pltpu.sync_copy(q_hbm.at[pl.ds(qbase + step * _IDX_ROWS, _IDX_ROWS)],
                            idx_v)
            for j in range(_IDX_ROWS):
                pltpu.async_copy(
                    t_hbm.at[idx_v.at[j]],
                    rows_v.at[pl.ds(j * _IDX_MINOR, _IDX_MINOR)],
                    sem,
                )

        def drain(idx_v, rows_v, sem):
            for j in range(_IDX_ROWS):
                pltpu.make_async_copy(
                    t_hbm.at[idx_v.at[j]],
                    rows_v.at[pl.ds(j * _IDX_MINOR, _IDX_MINOR)],
                    sem,
                ).wait()

        def compute(step, rows_v):
            for b0 in range(0, _CB, 4):
                def body(l, accs):
                    ws = [w_v[l, pl.ds(16 * k, 16)] for k in range(4)]
                    nxt = []
                    for g in range(4):
                        r = (b0 + g) * _SLEN + l
                        for k in range(4):
                            nxt.append(accs[g * 4 + k]
                                       + rows_v[r, pl.ds(16 * k, 16)] * ws[k])
                    return tuple(nxt)

                zero = jnp.zeros((16,), jnp.float32)
                accs = lax.fori_loop(0, _SLEN, body, (zero,) * 16)
                for g in range(4):
                    for k in range(4):
                        out_v[b0 + g, pl.ds(16 * k, 16)] = accs[g * 4 + k]
            pltpu.sync_copy(out_v,
                            o_hbm.at[pl.ds(wid * _BPW + step * _CB, _CB)])

        fire(jnp.int32(0), idx0, rows0, sem0)

        @pl.loop(0, _STEPS, step=2)
        def _(s0):
            fire(s0 + 1, idx1, rows1, sem1)
            drain(idx0, rows0, sem0)
            compute(s0, rows0)

            @pl.when(s0 + 2 < _STEPS)
            def _():
                fire(s0 + 2, idx0, rows0, sem0)

            drain(idx1, rows1, sem1)
            compute(s0 + 1, rows1)

    return run(q2, table_rm, w)
